# baseline (speedup 1.0000x reference)
import jax
import jax.numpy as jnp
from jax import lax
from jax.experimental import pallas as pl
from jax.experimental.pallas import tpu as pltpu
from jax.experimental.pallas import tpu_sc as plsc

NC = 2
NS = 16
CH = 128
DH = 64


def _sc_segment_sum(fs_a, fs_b, src_t, dst_t, zrow, zhist, acc_rows, k_chunks):
    share = acc_rows // NS

    def body(fsa_hbm, fsb_hbm, src_hbm, dst_hbm, zrow_hbm, zhist_hbm,
             sums_hbm, degs_hbm,
             src_v, dst_v, rows_a, hist_v, gsem_a, acc):
        c = lax.axis_index("c")
        s = lax.axis_index("s")
        ones16 = jnp.ones((16,), jnp.float32)
        base = s * share

        cp1 = pltpu.async_copy(src_hbm.at[s], src_v, gsem_a)
        cp2 = pltpu.async_copy(dst_hbm.at[s], dst_v, gsem_a)
        cp3 = pltpu.async_copy(zhist_hbm, hist_v, gsem_a)
        pltpu.sync_copy(zrow_hbm, acc.at[pl.ds(base, share)])
        cp3.wait()
        cp2.wait()
        cp1.wait()
        plsc.subcore_barrier()

        def accumulate(feat_hbm, with_hist):
            def chunk(j, carry):
                cp = pltpu.async_copy(feat_hbm.at[src_v.at[j]], rows_a, gsem_a)
                if with_hist:
                    for l in range(CH // 16):
                        idx = dst_v[j, pl.ds(l * 16, 16)]
                        plsc.addupdate_scatter(hist_v, [idx], ones16)
                cp.wait()
                pltpu.sync_copy(rows_a, acc.at[dst_v.at[j]], add=True)
                return carry

            lax.fori_loop(0, k_chunks, chunk, 0)

        @pl.when(c == 0)
        def _():
            accumulate(fsa_hbm, with_hist=True)
            pltpu.sync_copy(hist_v, degs_hbm.at[s])

        @pl.when(c == 1)
        def _():
            accumulate(fsb_hbm, with_hist=False)

        plsc.subcore_barrier()
        pltpu.sync_copy(acc.at[pl.ds(base, share)],
                        sums_hbm.at[c, pl.ds(base, share)])

    mesh = plsc.VectorSubcoreMesh(core_axis_name="c", subcore_axis_name="s",
                                  num_cores=NC, num_subcores=NS)
    fn = pl.kernel(
        body,
        out_type=(jax.ShapeDtypeStruct((NC, acc_rows, DH), jnp.float32),
                  jax.ShapeDtypeStruct((NS, acc_rows), jnp.float32)),
        mesh=mesh,
        compiler_params=pltpu.CompilerParams(use_tc_tiling_on_sc=False,
                                             needs_layout_passes=False),
        scratch_types=[
            pltpu.VMEM((k_chunks, CH), jnp.int32),
            pltpu.VMEM((k_chunks, CH), jnp.int32),
            pltpu.VMEM((CH, DH), jnp.float32),
            pltpu.VMEM((acc_rows,), jnp.float32),
            pltpu.SemaphoreType.DMA,
            pltpu.VMEM_SHARED((acc_rows, DH), jnp.float32),
        ],
    )
    return fn(fs_a, fs_b, src_t, dst_t, zrow, zhist)


def _tc_combine(sums, degs, fd, wst, wnt_a, wnt_b, bs, bn, rows, blk):
    d = fd.shape[1]
    ones16 = jnp.ones((NS, 1), jnp.float32)

    def body(p_ref, d_ref, o16_ref, fd_ref, wst_ref, wa_ref, wb_ref, bs_ref,
             bn_ref, o_ref):
        sa = p_ref[0]
        sb = p_ref[1]
        deg = jnp.dot(d_ref[...], o16_ref[...],
                      preferred_element_type=jnp.float32)
        inv = 1.0 / jnp.maximum(deg, 1.0)
        o_ref[...] = (
            jnp.dot(fd_ref[...], wst_ref[...], preferred_element_type=jnp.float32)
            + jnp.dot(sa * inv, wa_ref[...], preferred_element_type=jnp.float32)
            + jnp.dot(sb * inv, wb_ref[...], preferred_element_type=jnp.float32)
            + bs_ref[...] + bn_ref[...]
        )

    return pl.pallas_call(
        body,
        grid=(rows // blk,),
        in_specs=[
            pl.BlockSpec((2, blk, DH), lambda i: (0, i, 0)),
            pl.BlockSpec((blk, NS), lambda i: (i, 0)),
            pl.BlockSpec((NS, 1), lambda i: (0, 0)),
            pl.BlockSpec((blk, d), lambda i: (i, 0)),
            pl.BlockSpec((d, d), lambda i: (0, 0)),
            pl.BlockSpec((DH, d), lambda i: (0, 0)),
            pl.BlockSpec((DH, d), lambda i: (0, 0)),
            pl.BlockSpec((1, d), lambda i: (0, 0)),
            pl.BlockSpec((1, d), lambda i: (0, 0)),
        ],
        out_specs=pl.BlockSpec((blk, d), lambda i: (i, 0)),
        out_shape=jax.ShapeDtypeStruct((rows, d), jnp.float32),
    )(sums, degs, ones16, fd, wst, wnt_a, wnt_b, bs, bn)


def kernel(feat_src, feat_dst, edge_index, W_self, b_self, W_neigh, b_neigh):
    n_src, d = feat_src.shape
    n_dst = feat_dst.shape[0]
    e = edge_index.shape[1]

    k_chunks = -(-e // (NS * CH))
    e_pad = NS * k_chunks * CH
    acc_rows = -(-(n_dst + 128) // 512) * 512

    src = edge_index[0].astype(jnp.int32)
    dst = edge_index[1].astype(jnp.int32)
    pad = e_pad - e
    pad_cycle = jnp.arange(pad, dtype=jnp.int32) % 128
    src_t = jnp.concatenate([src, pad_cycle]).reshape(NS, k_chunks, CH)
    dst_t = jnp.concatenate([dst, n_dst + pad_cycle]).reshape(NS, k_chunks, CH)

    fs_a = feat_src[:, :DH]
    fs_b = feat_src[:, DH:]
    zrow = jnp.zeros((acc_rows // NS, DH), jnp.float32)
    zhist = jnp.zeros((acc_rows,), jnp.float32)

    sums, degs = _sc_segment_sum(fs_a, fs_b, src_t, dst_t, zrow, zhist,
                                 acc_rows, k_chunks)

    wnt = W_neigh.T
    return _tc_combine(sums, degs.T, feat_dst, W_self.T, wnt[:DH], wnt[DH:],
                       b_self.reshape(1, d), b_neigh.reshape(1, d),
                       n_dst, 1000)
